# baseline (speedup 1.0000x reference)
import functools

import jax
import jax.numpy as jnp
from jax import lax
from jax.experimental import pallas as pl
from jax.experimental.pallas import tpu as pltpu
from jax.experimental.pallas import tpu_sc as plsc

EMBED = 64
GRP = 128
NK = 4
CHUNK = NK * GRP


@functools.lru_cache(maxsize=None)
def _make_gather(B):
    info = plsc.get_sparse_core_info()
    nc, ns = info.num_cores, info.num_subcores
    nw = nc * ns
    rows_per_worker = B // nw
    assert rows_per_worker % CHUNK == 0
    nchunk = rows_per_worker // CHUNK
    mesh = plsc.VectorSubcoreMesh(core_axis_name="c", subcore_axis_name="s")

    @functools.partial(
        pl.kernel,
        mesh=mesh,
        out_type=jax.ShapeDtypeStruct((B, EMBED), jnp.float32),
        scratch_types=[
            pltpu.VMEM((NK, GRP), jnp.int32),
            pltpu.VMEM((CHUNK, EMBED), jnp.float32),
            pltpu.SemaphoreType.DMA,
        ],
        compiler_params=pltpu.CompilerParams(use_tc_tiling_on_sc=False),
    )
    def gather_kernel(table_hbm, idx_hbm, out_hbm, idx_v, rows_v, sem):
        wid = lax.axis_index("s") * nc + lax.axis_index("c")

        def chunk_body(g, carry):
            irow = wid * (rows_per_worker // GRP) + g * NK
            base = wid * rows_per_worker + g * CHUNK
            pltpu.sync_copy(idx_hbm.at[pl.ds(irow, NK)], idx_v)
            cps = [
                pltpu.async_copy(
                    table_hbm.at[idx_v.at[j]],
                    rows_v.at[pl.ds(j * GRP, GRP)],
                    sem,
                )
                for j in range(NK)
            ]
            for cp in cps:
                cp.wait()
            pltpu.sync_copy(rows_v, out_hbm.at[pl.ds(base, CHUNK)])
            return carry

        lax.fori_loop(0, nchunk, chunk_body, 0)

    return gather_kernel


def kernel(indices, weight):
    b, h = indices.shape
    flat = b * h
    idx2d = indices.reshape(flat // GRP, GRP)
    out = _make_gather(flat)(weight, idx2d)
    return out.reshape(b, h, EMBED)


# speedup vs baseline: 1.0430x; 1.0430x over previous
import functools

import jax
import jax.numpy as jnp
from jax import lax
from jax.experimental import pallas as pl
from jax.experimental.pallas import tpu as pltpu
from jax.experimental.pallas import tpu_sc as plsc

EMBED = 64
GRP = 128
NK = 4
CHUNK = NK * GRP


@functools.lru_cache(maxsize=None)
def _make_gather(B):
    info = plsc.get_sparse_core_info()
    nc, ns = info.num_cores, info.num_subcores
    nw = nc * ns
    rpw = B // nw
    assert rpw % CHUNK == 0
    nchunk = rpw // CHUNK
    assert nchunk % 2 == 0
    irows = rpw // GRP
    mesh = plsc.VectorSubcoreMesh(core_axis_name="c", subcore_axis_name="s")

    @functools.partial(
        pl.kernel,
        mesh=mesh,
        out_type=jax.ShapeDtypeStruct((B, EMBED), jnp.float32),
        scratch_types=[
            pltpu.VMEM((irows, GRP), jnp.int32),
            pltpu.VMEM((2, CHUNK, EMBED), jnp.float32),
            pltpu.SemaphoreType.DMA,
            pltpu.SemaphoreType.DMA,
        ],
        compiler_params=pltpu.CompilerParams(use_tc_tiling_on_sc=False),
    )
    def gather_kernel(table_hbm, idx_hbm, out_hbm, idx_v, rows_v, gsem0, gsem1):
        wid = lax.axis_index("s") * nc + lax.axis_index("c")
        gsem = (gsem0, gsem1)

        pltpu.sync_copy(idx_hbm.at[pl.ds(wid * irows, irows)], idx_v)

        def fire_gathers(t, b):
            for j in range(NK):
                pltpu.async_copy(
                    table_hbm.at[idx_v.at[t * NK + j]],
                    rows_v.at[b, pl.ds(j * GRP, GRP)],
                    gsem[b],
                )

        def drain_gathers(b):
            pltpu.make_async_copy(
                out_hbm.at[pl.ds(0, CHUNK)], rows_v.at[b], gsem[b]
            ).wait()

        fire_gathers(0, 0)
        fire_gathers(1, 1)

        def outer(t2, carry):
            for b in range(2):
                t = t2 * 2 + b
                drain_gathers(b)
                wcp = pltpu.make_async_copy(
                    rows_v.at[b],
                    out_hbm.at[pl.ds((wid * nchunk + t) * CHUNK, CHUNK)],
                    gsem[b],
                )
                wcp.start()
                wcp.wait()

                @pl.when(t2 < nchunk // 2 - 1)
                def _():
                    fire_gathers(t + 2, b)

            return carry

        lax.fori_loop(0, nchunk // 2, outer, 0)

    return gather_kernel


def kernel(indices, weight):
    b, h = indices.shape
    flat = b * h
    idx2d = indices.reshape(flat // GRP, GRP)
    out = _make_gather(flat)(weight, idx2d)
    return out.reshape(b, h, EMBED)


# speedup vs baseline: 1.0436x; 1.0006x over previous
import functools

import jax
import jax.numpy as jnp
from jax import lax
from jax.experimental import pallas as pl
from jax.experimental.pallas import tpu as pltpu
from jax.experimental.pallas import tpu_sc as plsc

EMBED = 64
GRP = 512
NK = 1
CHUNK = NK * GRP


@functools.lru_cache(maxsize=None)
def _make_gather(B):
    info = plsc.get_sparse_core_info()
    nc, ns = info.num_cores, info.num_subcores
    nw = nc * ns
    rpw = B // nw
    assert rpw % CHUNK == 0
    nchunk = rpw // CHUNK
    assert nchunk % 2 == 0
    irows = rpw // GRP
    mesh = plsc.VectorSubcoreMesh(core_axis_name="c", subcore_axis_name="s")

    @functools.partial(
        pl.kernel,
        mesh=mesh,
        out_type=jax.ShapeDtypeStruct((B, EMBED), jnp.float32),
        scratch_types=[
            pltpu.VMEM((irows, GRP), jnp.int32),
            pltpu.VMEM((2, CHUNK, EMBED), jnp.float32),
            pltpu.SemaphoreType.DMA,
            pltpu.SemaphoreType.DMA,
        ],
        compiler_params=pltpu.CompilerParams(use_tc_tiling_on_sc=False),
    )
    def gather_kernel(table_hbm, idx_hbm, out_hbm, idx_v, rows_v, gsem0, gsem1):
        wid = lax.axis_index("s") * nc + lax.axis_index("c")
        gsem = (gsem0, gsem1)

        pltpu.sync_copy(idx_hbm.at[pl.ds(wid * irows, irows)], idx_v)

        def fire_gathers(t, b):
            for j in range(NK):
                pltpu.async_copy(
                    table_hbm.at[idx_v.at[t * NK + j]],
                    rows_v.at[b, pl.ds(j * GRP, GRP)],
                    gsem[b],
                )

        def drain_gathers(b):
            pltpu.make_async_copy(
                out_hbm.at[pl.ds(0, CHUNK)], rows_v.at[b], gsem[b]
            ).wait()

        fire_gathers(0, 0)
        fire_gathers(1, 1)

        def outer(t2, carry):
            for b in range(2):
                t = t2 * 2 + b
                drain_gathers(b)
                wcp = pltpu.make_async_copy(
                    rows_v.at[b],
                    out_hbm.at[pl.ds((wid * nchunk + t) * CHUNK, CHUNK)],
                    gsem[b],
                )
                wcp.start()
                wcp.wait()

                @pl.when(t2 < nchunk // 2 - 1)
                def _():
                    fire_gathers(t + 2, b)

            return carry

        lax.fori_loop(0, nchunk // 2, outer, 0)

    return gather_kernel


def kernel(indices, weight):
    b, h = indices.shape
    flat = b * h
    idx2d = indices.reshape(flat // GRP, GRP)
    out = _make_gather(flat)(weight, idx2d)
    return out.reshape(b, h, EMBED)
